# baseline (speedup 1.0000x reference)
import functools

import jax
import jax.numpy as jnp
from jax import lax
from jax.experimental import pallas as pl
from jax.experimental.pallas import tpu as pltpu
from jax.experimental.pallas import tpu_sc as plsc

N = 10000
E = 320000
D = 128
H = 64
NPO = 5000
NLEVELS = 4

NC = 2
NS = 16
NW = NC * NS

C = 128
NCH = 80
EPAD = NW * NCH * C
ROWS_PER_TILE = 640
NPAD = NS * ROWS_PER_TILE

RB = 1000
C_PO = 80
NCH_PO = 2
PO_PAD = NW * NCH_PO * C_PO

EPW = NCH * C
IB = 8
NCH_CAP = 88
CAP = NCH_CAP * C
NBLK_PRE = NCH // IB

_mesh = plsc.VectorSubcoreMesh(core_axis_name="c", subcore_axis_name="s",
                               num_cores=NC, num_subcores=NS)


def _mlp_block(x, w1_ref, b1_ref, w2_ref, b2_ref, w3_ref, b3_ref):
    y = jnp.dot(x, w1_ref[...], preferred_element_type=jnp.float32) + b1_ref[...]
    y = jnp.where(y > 0, y, 0.01 * y)
    y = jnp.dot(y, w2_ref[...], preferred_element_type=jnp.float32) + b2_ref[...]
    y = jnp.where(y > 0, y, 0.01 * y)
    return jnp.dot(y, w3_ref[...], preferred_element_type=jnp.float32) + b3_ref[...]


def _full_spec(shape):
    return pl.BlockSpec(shape, lambda i: (0, 0))


def _gidx_body(src_ref, r_ref, o_ref):
    o_ref[...] = src_ref[...] + N * r_ref[...]


def _build_gidx(src_pad, r_pad):
    x = src_pad.reshape(EPAD // 128, 128)
    y = r_pad.reshape(EPAD // 128, 128)
    out = pl.pallas_call(
        _gidx_body,
        out_shape=jax.ShapeDtypeStruct((EPAD // 128, 128), jnp.int32),
    )(x, y)
    return out.reshape(NW, NCH, C)


def _table_body(h_ref, w1, b1, w2, b2, w3, b3, t_ref):
    i = pl.program_id(0)

    @pl.when(i < N // RB)
    def _():
        t_ref[...] = h_ref[...]

    @pl.when(i >= N // RB)
    def _():
        t_ref[...] = _mlp_block(h_ref[...], w1, b1, w2, b2, w3, b3)


def _build_table(hcur, w1, b1, w2, b2, w3, b3):
    nb = N // RB
    return pl.pallas_call(
        _table_body,
        grid=(2 * nb,),
        in_specs=[
            pl.BlockSpec((RB, D), lambda i: (i % (N // RB), 0)),
            _full_spec((D, H)), _full_spec((1, H)),
            _full_spec((H, H)), _full_spec((1, H)),
            _full_spec((H, D)), _full_spec((1, D)),
        ],
        out_specs=pl.BlockSpec((RB, D), lambda i: (i, 0)),
        out_shape=jax.ShapeDtypeStruct((2 * N, D), jnp.float32),
    )(hcur, w1, b1, w2, b2, w3, b3)


def _pre_body(gidx_hbm, dst_hbm, nl_hbm, zeros16_hbm, ones_hbm,
              eidx_hbm, edst_hbm, cnts_hbm, degp_hbm,
              gbuf, dbuf, nlbuf, e1, d1, e2, d2, e3, d3, cbuf, ones_v, dacc,
              isem):
    c = lax.axis_index("c")
    s = lax.axis_index("s")
    w = s * NC + c
    rbase = s * ROWS_PER_TILE
    pltpu.sync_copy(nl_hbm, nlbuf)
    pltpu.sync_copy(ones_hbm, ones_v)
    pltpu.sync_copy(zeros16_hbm.at[pl.ds(rbase, ROWS_PER_TILE)],
                    dacc.at[pl.ds(rbase, ROWS_PER_TILE)])
    plsc.subcore_barrier()
    lists = ((e1, d1), (e2, d2), (e3, d3))

    def do_block(slot, cnts):
        new = list(cnts)
        for jj in range(IB):
            pltpu.sync_copy(ones_v, dacc.at[dbuf.at[slot, jj]], add=True)
            for j in range(C // 16):
                dv = dbuf[slot, jj, pl.ds(j * 16, 16)]
                gv = gbuf[slot, jj, pl.ds(j * 16, 16)]
                lv = plsc.load_gather(nlbuf, [dv])
                for li in range(3):
                    eref, dref = lists[li]
                    m = lv == (li + 1)
                    plsc.store_compressed(eref.at[pl.ds(new[li], 16)], gv,
                                          mask=m)
                    plsc.store_compressed(dref.at[pl.ds(new[li], 16)], dv,
                                          mask=m)
                    new[li] = new[li] + jnp.sum(m.astype(jnp.int32))
        return tuple(new)

    def load_blk(b, slot):
        pltpu.async_copy(gidx_hbm.at[w, pl.ds(b * IB, IB)], gbuf.at[slot],
                         isem.at[slot])
        pltpu.async_copy(dst_hbm.at[w, pl.ds(b * IB, IB)], dbuf.at[slot],
                         isem.at[slot])

    def wait_blk(b, slot):
        pltpu.make_async_copy(gidx_hbm.at[w, pl.ds(b * IB, IB)],
                              gbuf.at[slot], isem.at[slot]).wait()
        pltpu.make_async_copy(dst_hbm.at[w, pl.ds(b * IB, IB)],
                              dbuf.at[slot], isem.at[slot]).wait()

    pltpu.sync_copy(gidx_hbm.at[w, pl.ds(0, IB)], gbuf.at[0])
    pltpu.sync_copy(dst_hbm.at[w, pl.ds(0, IB)], dbuf.at[0])
    load_blk(1, 1)

    def pair(p, cnts):
        b0 = 2 * p

        @pl.when(p > 0)
        def _():
            wait_blk(b0, 0)

        cnts = do_block(0, cnts)

        @pl.when(b0 + 2 < NBLK_PRE)
        def _():
            load_blk(b0 + 2, 0)

        wait_blk(b0 + 1, 1)
        cnts = do_block(1, cnts)

        @pl.when(b0 + 3 < NBLK_PRE)
        def _():
            load_blk(b0 + 3, 1)

        return cnts

    z = jnp.int32(0)
    cnts = lax.fori_loop(0, NBLK_PRE // 2, pair, (z, z, z))

    tg = jnp.zeros((16,), jnp.int32)
    td = jnp.full((16,), N, jnp.int32)
    mall = jnp.ones((16,), jnp.bool_)
    for li in range(3):
        eref, dref = lists[li]
        for j in range(C // 16):
            plsc.store_compressed(eref.at[pl.ds(cnts[li] + j * 16, 16)],
                                  tg, mask=mall)
            plsc.store_compressed(dref.at[pl.ds(cnts[li] + j * 16, 16)],
                                  td, mask=mall)
        cbuf[...] = jnp.zeros((16,), jnp.int32) + cnts[li]
        pltpu.sync_copy(cbuf, cnts_hbm.at[li, w])
        pltpu.sync_copy(eref, eidx_hbm.at[li, w])
        pltpu.sync_copy(dref, edst_hbm.at[li, w])
    plsc.subcore_barrier()
    pltpu.sync_copy(dacc.at[pl.ds(rbase, ROWS_PER_TILE)],
                    degp_hbm.at[c, pl.ds(rbase, ROWS_PER_TILE)])


_sc_preprocess = pl.kernel(
    _pre_body,
    out_type=[
        jax.ShapeDtypeStruct((3, NW, CAP), jnp.int32),
        jax.ShapeDtypeStruct((3, NW, CAP), jnp.int32),
        jax.ShapeDtypeStruct((3, NW, 16), jnp.int32),
        jax.ShapeDtypeStruct((NC, NPAD, 16), jnp.float32),
    ],
    mesh=_mesh,
    scratch_types=[
        pltpu.VMEM((2, IB, C), jnp.int32),
        pltpu.VMEM((2, IB, C), jnp.int32),
        pltpu.VMEM((N + 16,), jnp.int32),
        pltpu.VMEM((CAP,), jnp.int32), pltpu.VMEM((CAP,), jnp.int32),
        pltpu.VMEM((CAP,), jnp.int32), pltpu.VMEM((CAP,), jnp.int32),
        pltpu.VMEM((CAP,), jnp.int32), pltpu.VMEM((CAP,), jnp.int32),
        pltpu.VMEM((16,), jnp.int32),
        pltpu.VMEM((C, 16), jnp.float32),
        pltpu.VMEM_SHARED((NPAD, 16), jnp.float32),
        pltpu.SemaphoreType.DMA((2,)),
    ],
    compiler_params=pltpu.CompilerParams(use_tc_tiling_on_sc=False,
                                        needs_layout_passes=False))


def _sc_body(gidx_hbm, dst_hbm, cnt_hbm, table_hbm, zeros_hbm,
             out_hbm, gbuf, dbuf, rows, acc, gsem, isem, ssem, cbuf):
    c = lax.axis_index("c")
    s = lax.axis_index("s")
    w = s * NC + c
    rbase = s * ROWS_PER_TILE
    pltpu.sync_copy(cnt_hbm.at[w], cbuf)
    cnt = jnp.max(cbuf[...])
    nch = jnp.maximum(lax.div(cnt + (C - 1), C), 1)
    nblk = lax.div(nch + (IB - 1), IB)
    pltpu.sync_copy(zeros_hbm.at[pl.ds(rbase, ROWS_PER_TILE)],
                    acc.at[pl.ds(rbase, ROWS_PER_TILE)])
    plsc.subcore_barrier()

    pltpu.sync_copy(gidx_hbm.at[w, pl.ds(0, IB)], gbuf.at[0])
    pltpu.sync_copy(dst_hbm.at[w, pl.ds(0, IB)], dbuf.at[0])

    def issue(gslot, bslot, jj):
        pltpu.async_copy(table_hbm.at[gbuf.at[bslot, jj]], rows.at[gslot],
                         gsem.at[gslot])

    def blk_body(b, carry):
        bslot = lax.rem(b, 2)
        base = b * IB

        @pl.when(b > 0)
        def _():
            pltpu.make_async_copy(gidx_hbm.at[w, pl.ds(base, IB)],
                                  gbuf.at[bslot], isem.at[bslot]).wait()
            pltpu.make_async_copy(dst_hbm.at[w, pl.ds(base, IB)],
                                  dbuf.at[bslot], isem.at[bslot]).wait()

        @pl.when(b + 1 < nblk)
        def _():
            nbase = (b + 1) * IB
            pltpu.async_copy(gidx_hbm.at[w, pl.ds(nbase, IB)],
                             gbuf.at[1 - bslot], isem.at[1 - bslot])
            pltpu.async_copy(dst_hbm.at[w, pl.ds(nbase, IB)],
                             dbuf.at[1 - bslot], isem.at[1 - bslot])

        def scat_desc(gslot, jj):
            return pltpu.make_async_copy(rows.at[gslot],
                                         acc.at[dbuf.at[bslot, jj]],
                                         ssem.at[gslot])

        issue(lax.rem(base, 2), bslot, 0)
        for jj in range(IB):
            i = base + jj
            gslot = lax.rem(i, 2)

            @pl.when(i < nch)
            def _(jj=jj, i=i, gslot=gslot):
                pltpu.make_async_copy(table_hbm.at[gbuf.at[bslot, jj]],
                                      rows.at[gslot], gsem.at[gslot]).wait()
                if jj + 1 < IB:
                    @pl.when(i + 1 < nch)
                    def _():
                        if jj >= 1:
                            scat_desc(1 - gslot, jj - 1).wait()
                        issue(1 - gslot, bslot, jj + 1)
                pltpu.async_copy(rows.at[gslot], acc.at[dbuf.at[bslot, jj]],
                                 ssem.at[gslot], add=True)
        for jj in range(IB):
            i = base + jj
            gslot = lax.rem(i, 2)

            @pl.when(jnp.logical_and(
                i < nch,
                jnp.logical_or(jj >= IB - 2, i + 2 >= nch)))
            def _(jj=jj, gslot=gslot):
                scat_desc(gslot, jj).wait()
        return carry

    lax.fori_loop(0, nblk, blk_body, 0)
    plsc.subcore_barrier()
    pltpu.sync_copy(acc.at[pl.ds(rbase, ROWS_PER_TILE)],
                    out_hbm.at[c, pl.ds(rbase, ROWS_PER_TILE)])


_sc_scatter = pl.kernel(
    _sc_body,
    out_type=[jax.ShapeDtypeStruct((NC, NPAD, D), jnp.float32)],
    mesh=_mesh,
    scratch_types=[
        pltpu.VMEM((2, IB, C), jnp.int32),
        pltpu.VMEM((2, IB, C), jnp.int32),
        pltpu.VMEM((2, C, D), jnp.float32),
        pltpu.VMEM_SHARED((NPAD, D), jnp.float32),
        pltpu.SemaphoreType.DMA((2,)),
        pltpu.SemaphoreType.DMA((2,)),
        pltpu.SemaphoreType.DMA((2,)),
        pltpu.VMEM((16,), jnp.int32),
    ],
    compiler_params=pltpu.CompilerParams(use_tc_tiling_on_sc=False,
                                        needs_layout_passes=False))


def _po_body(hfin_hbm, poidx_hbm, out_hbm, pall, rows):
    c = lax.axis_index("c")
    s = lax.axis_index("s")
    w = s * NC + c
    pltpu.sync_copy(poidx_hbm.at[w], pall)
    for j in range(NCH_PO):
        pltpu.sync_copy(hfin_hbm.at[pall.at[j]], rows)
        pltpu.sync_copy(rows, out_hbm.at[pl.ds(w * NCH_PO * C_PO + j * C_PO,
                                               C_PO)])


_po_gather = pl.kernel(
    _po_body,
    out_type=jax.ShapeDtypeStruct((PO_PAD, D), jnp.float32),
    mesh=_mesh,
    scratch_types=[
        pltpu.VMEM((NCH_PO, C_PO), jnp.int32),
        pltpu.VMEM((C_PO, D), jnp.float32),
    ],
    compiler_params=pltpu.CompilerParams(use_tc_tiling_on_sc=False,
                                        needs_layout_passes=False),
)


def _update_body(l, acc0_ref, acc1_ref, deg0_ref, deg1_ref, lvl_ref, h_ref,
                 w1, b1, w2, b2, w3, b3, o_ref):
    i = pl.program_id(0)
    d = (deg0_ref[pl.ds(i * RB, RB), :] + deg1_ref[pl.ds(i * RB, RB), :])
    degcol = jnp.maximum(d[:, 0:1], 1.0)
    neigh = (acc0_ref[...] + acc1_ref[...]) / degcol
    upd = _mlp_block(neigh, w1, b1, w2, b2, w3, b3)
    lvl = lvl_ref[pl.ds(i * RB, RB), :]
    o_ref[...] = jnp.where(lvl == l, upd, h_ref[...])


def _update(l, acc0, acc1, deg0, deg1, lvl2d, hcur, w1, b1, w2, b2, w3, b3):
    return pl.pallas_call(
        functools.partial(_update_body, l),
        grid=(N // RB,),
        in_specs=[
            pl.BlockSpec((RB, D), lambda i: (i, 0)),
            pl.BlockSpec((RB, D), lambda i: (i, 0)),
            _full_spec((NPAD, 16)),
            _full_spec((NPAD, 16)),
            _full_spec((N, 1)),
            pl.BlockSpec((RB, D), lambda i: (i, 0)),
            _full_spec((D, H)), _full_spec((1, H)),
            _full_spec((H, H)), _full_spec((1, H)),
            _full_spec((H, D)), _full_spec((1, D)),
        ],
        out_specs=pl.BlockSpec((RB, D), lambda i: (i, 0)),
        out_shape=jax.ShapeDtypeStruct((N, D), jnp.float32),
    )(acc0, acc1, deg0, deg1, lvl2d, hcur, w1, b1, w2, b2, w3, b3)


def _upd_tab_body(l, acc0_ref, acc1_ref, deg0_ref, deg1_ref, lvl_ref, h_ref,
                  w1, b1, w2, b2, w3, b3, fw1, fb1, fw2, fb2, fw3, fb3,
                  o_ref, t_ref):
    i = pl.program_id(0)
    nb = N // RB
    ii = lax.rem(i, nb)
    d = (deg0_ref[pl.ds(ii * RB, RB), :] + deg1_ref[pl.ds(ii * RB, RB), :])
    degcol = jnp.maximum(d[:, 0:1], 1.0)
    neigh = (acc0_ref[...] + acc1_ref[...]) / degcol
    upd = _mlp_block(neigh, w1, b1, w2, b2, w3, b3)
    lvl = lvl_ref[pl.ds(ii * RB, RB), :]
    hnew = jnp.where(lvl == l, upd, h_ref[...])
    o_ref[...] = hnew

    @pl.when(i < nb)
    def _():
        t_ref[...] = hnew

    @pl.when(i >= nb)
    def _():
        t_ref[...] = _mlp_block(hnew, fw1, fb1, fw2, fb2, fw3, fb3)


def _update_and_table(l, acc0, acc1, deg0, deg1, lvl2d, hcur,
                      w1, b1, w2, b2, w3, b3, fw1, fb1, fw2, fb2, fw3, fb3):
    nb = N // RB
    blk = pl.BlockSpec((RB, D), lambda i: (lax.rem(i, N // RB), 0))
    return pl.pallas_call(
        functools.partial(_upd_tab_body, l),
        grid=(2 * nb,),
        in_specs=[
            blk, blk,
            _full_spec((NPAD, 16)), _full_spec((NPAD, 16)),
            _full_spec((N, 1)),
            blk,
            _full_spec((D, H)), _full_spec((1, H)),
            _full_spec((H, H)), _full_spec((1, H)),
            _full_spec((H, D)), _full_spec((1, D)),
            _full_spec((D, H)), _full_spec((1, H)),
            _full_spec((H, H)), _full_spec((1, H)),
            _full_spec((H, D)), _full_spec((1, D)),
        ],
        out_specs=[
            pl.BlockSpec((RB, D), lambda i: (lax.rem(i, N // RB), 0)),
            pl.BlockSpec((RB, D), lambda i: (i, 0)),
        ],
        out_shape=[
            jax.ShapeDtypeStruct((N, D), jnp.float32),
            jax.ShapeDtypeStruct((2 * N, D), jnp.float32),
        ],
    )(acc0, acc1, deg0, deg1, lvl2d, hcur, w1, b1, w2, b2, w3, b3,
      fw1, fb1, fw2, fb2, fw3, fb3)


def _pr_body(x_ref, w1, b1, w2, b2, w3, b3, o_ref):
    o_ref[...] = _mlp_block(x_ref[...], w1, b1, w2, b2, w3, b3)


def _pr_mlp(x, w1, b1, w2, b2, w3, b3):
    rb = PO_PAD // 8
    return pl.pallas_call(
        _pr_body,
        grid=(8,),
        in_specs=[
            pl.BlockSpec((rb, D), lambda i: (i, 0)),
            _full_spec((D, H)), _full_spec((1, H)),
            _full_spec((H, H)), _full_spec((1, H)),
            _full_spec((H, D)), _full_spec((1, D)),
        ],
        out_specs=pl.BlockSpec((rb, D), lambda i: (i, 0)),
        out_shape=jax.ShapeDtypeStruct((PO_PAD, D), jnp.float32),
    )(x, w1, b1, w2, b2, w3, b3)


def kernel(h, edge_index, edge_r, node_level, PO_mask,
           fi_w1, fi_b1, fi_w2, fi_b2, fi_w3, fi_b3,
           fa_w1, fa_b1, fa_w2, fa_b2, fa_w3, fa_b3,
           pr_w1, pr_b1, pr_w2, pr_b2, pr_w3, pr_b3):
    src = edge_index[0]
    dst = edge_index[1]
    pad = EPAD - E
    src_p = jnp.concatenate([src, jnp.zeros((pad,), jnp.int32)])
    r_p = jnp.concatenate([edge_r, jnp.zeros((pad,), jnp.int32)])
    dst_p = jnp.concatenate([dst, jnp.full((pad,), N, jnp.int32)])
    gidx2 = _build_gidx(src_p, r_p)
    dst2 = dst_p.reshape(NW, NCH, C)
    zeros = jnp.zeros((NPAD, D), jnp.float32)
    zeros16 = jnp.zeros((NPAD, 16), jnp.float32)
    ones16 = jnp.ones((C, 16), jnp.float32)
    lvl2d = node_level.reshape(N, 1)
    nlpad = jnp.concatenate([node_level, jnp.zeros((16,), jnp.int32)])

    eidx3, edst3, cnts3, degp = _sc_preprocess(gidx2, dst2, nlpad, zeros16,
                                               ones16)
    deg0, deg1 = degp[0], degp[1]

    fi_b1r, fi_b2r, fi_b3r = (fi_b1.reshape(1, H), fi_b2.reshape(1, H),
                              fi_b3.reshape(1, D))
    fa_b1r, fa_b2r, fa_b3r = (fa_b1.reshape(1, H), fa_b2.reshape(1, H),
                              fa_b3.reshape(1, D))
    pr_b1r, pr_b2r, pr_b3r = (pr_b1.reshape(1, H), pr_b2.reshape(1, H),
                              pr_b3.reshape(1, D))

    hcur = h
    T = _build_table(hcur, fi_w1, fi_b1r, fi_w2, fi_b2r, fi_w3, fi_b3r)
    for l in range(1, NLEVELS):
        (parts,) = _sc_scatter(eidx3[l - 1].reshape(NW, NCH_CAP, C),
                               edst3[l - 1].reshape(NW, NCH_CAP, C),
                               cnts3[l - 1], T, zeros)
        if l < NLEVELS - 1:
            hcur, T = _update_and_table(
                l, parts[0], parts[1], deg0, deg1, lvl2d, hcur,
                fa_w1, fa_b1r, fa_w2, fa_b2r, fa_w3, fa_b3r,
                fi_w1, fi_b1r, fi_w2, fi_b2r, fi_w3, fi_b3r)
        else:
            hcur = _update(l, parts[0], parts[1], deg0, deg1, lvl2d, hcur,
                           fa_w1, fa_b1r, fa_w2, fa_b2r, fa_w3, fa_b3r)

    po_p = jnp.concatenate(
        [PO_mask, jnp.zeros((PO_PAD - NPO,), jnp.int32)]).reshape(
            NW, NCH_PO, C_PO)
    rows = _po_gather(hcur, po_p)
    out = _pr_mlp(rows, pr_w1, pr_b1r, pr_w2, pr_b2r, pr_w3, pr_b3r)
    return out[:NPO]


# speedup vs baseline: 1.0132x; 1.0132x over previous
import functools

import jax
import jax.numpy as jnp
from jax import lax
from jax.experimental import pallas as pl
from jax.experimental.pallas import tpu as pltpu
from jax.experimental.pallas import tpu_sc as plsc

N = 10000
E = 320000
D = 128
H = 64
NPO = 5000
NLEVELS = 4

NC = 2
NS = 16
NW = NC * NS

C = 128
NCH = 80
EPAD = NW * NCH * C
ROWS_PER_TILE = 640
NPAD = NS * ROWS_PER_TILE

RB = 1000
C_PO = 80
NCH_PO = 2
PO_PAD = NW * NCH_PO * C_PO

EPW = NCH * C
IB = 8
NCH_CAP = 88
CAP = NCH_CAP * C
NBLK_PRE = NCH // IB

_mesh = plsc.VectorSubcoreMesh(core_axis_name="c", subcore_axis_name="s",
                               num_cores=NC, num_subcores=NS)


def _mlp_block(x, w1_ref, b1_ref, w2_ref, b2_ref, w3_ref, b3_ref):
    y = jnp.dot(x, w1_ref[...], preferred_element_type=jnp.float32) + b1_ref[...]
    y = jnp.where(y > 0, y, 0.01 * y)
    y = jnp.dot(y, w2_ref[...], preferred_element_type=jnp.float32) + b2_ref[...]
    y = jnp.where(y > 0, y, 0.01 * y)
    return jnp.dot(y, w3_ref[...], preferred_element_type=jnp.float32) + b3_ref[...]


def _full_spec(shape):
    return pl.BlockSpec(shape, lambda i: (0, 0))


def _gidx_body(src_ref, r_ref, o_ref):
    o_ref[...] = src_ref[...] + N * r_ref[...]


def _build_gidx(src_pad, r_pad):
    x = src_pad.reshape(EPAD // 128, 128)
    y = r_pad.reshape(EPAD // 128, 128)
    out = pl.pallas_call(
        _gidx_body,
        out_shape=jax.ShapeDtypeStruct((EPAD // 128, 128), jnp.int32),
    )(x, y)
    return out.reshape(NW, NCH, C)


def _table_body(h_ref, w1, b1, w2, b2, w3, b3, t_ref):
    i = pl.program_id(0)

    @pl.when(i < N // RB)
    def _():
        t_ref[...] = h_ref[...]

    @pl.when(i >= N // RB)
    def _():
        t_ref[...] = _mlp_block(h_ref[...], w1, b1, w2, b2, w3, b3)


def _build_table(hcur, w1, b1, w2, b2, w3, b3):
    nb = N // RB
    return pl.pallas_call(
        _table_body,
        grid=(2 * nb,),
        in_specs=[
            pl.BlockSpec((RB, D), lambda i: (i % (N // RB), 0)),
            _full_spec((D, H)), _full_spec((1, H)),
            _full_spec((H, H)), _full_spec((1, H)),
            _full_spec((H, D)), _full_spec((1, D)),
        ],
        out_specs=pl.BlockSpec((RB, D), lambda i: (i, 0)),
        out_shape=jax.ShapeDtypeStruct((2 * N, D), jnp.float32),
    )(hcur, w1, b1, w2, b2, w3, b3)


def _pre_body(gidx_hbm, dst_hbm, nl_hbm, zeros16_hbm, ones_hbm,
              eidx_hbm, edst_hbm, cnts_hbm, degp_hbm,
              gbuf, dbuf, nlbuf, e1, d1, e2, d2, e3, d3, cbuf, ones_v, dacc,
              isem, osem):
    c = lax.axis_index("c")
    s = lax.axis_index("s")
    w = s * NC + c
    rbase = s * ROWS_PER_TILE
    pltpu.sync_copy(nl_hbm, nlbuf)
    pltpu.sync_copy(ones_hbm, ones_v)
    pltpu.sync_copy(zeros16_hbm.at[pl.ds(rbase, ROWS_PER_TILE)],
                    dacc.at[pl.ds(rbase, ROWS_PER_TILE)])
    plsc.subcore_barrier()
    lists = ((e1, d1), (e2, d2), (e3, d3))

    def do_block(slot, cnts):
        new = list(cnts)
        descs = []
        for jj in range(IB):
            descs.append(pltpu.async_copy(ones_v, dacc.at[dbuf.at[slot, jj]],
                                          osem, add=True))
            for j in range(C // 16):
                dv = dbuf[slot, jj, pl.ds(j * 16, 16)]
                gv = gbuf[slot, jj, pl.ds(j * 16, 16)]
                lv = plsc.load_gather(nlbuf, [dv])
                for li in range(3):
                    eref, dref = lists[li]
                    m = lv == (li + 1)
                    plsc.store_compressed(eref.at[pl.ds(new[li], 16)], gv,
                                          mask=m)
                    plsc.store_compressed(dref.at[pl.ds(new[li], 16)], dv,
                                          mask=m)
                    new[li] = new[li] + jnp.sum(m.astype(jnp.int32))
        for dsc in descs:
            dsc.wait()
        return tuple(new)

    def load_blk(b, slot):
        pltpu.async_copy(gidx_hbm.at[w, pl.ds(b * IB, IB)], gbuf.at[slot],
                         isem.at[slot])
        pltpu.async_copy(dst_hbm.at[w, pl.ds(b * IB, IB)], dbuf.at[slot],
                         isem.at[slot])

    def wait_blk(b, slot):
        pltpu.make_async_copy(gidx_hbm.at[w, pl.ds(b * IB, IB)],
                              gbuf.at[slot], isem.at[slot]).wait()
        pltpu.make_async_copy(dst_hbm.at[w, pl.ds(b * IB, IB)],
                              dbuf.at[slot], isem.at[slot]).wait()

    pltpu.sync_copy(gidx_hbm.at[w, pl.ds(0, IB)], gbuf.at[0])
    pltpu.sync_copy(dst_hbm.at[w, pl.ds(0, IB)], dbuf.at[0])
    load_blk(1, 1)

    def pair(p, cnts):
        b0 = 2 * p

        @pl.when(p > 0)
        def _():
            wait_blk(b0, 0)

        cnts = do_block(0, cnts)

        @pl.when(b0 + 2 < NBLK_PRE)
        def _():
            load_blk(b0 + 2, 0)

        wait_blk(b0 + 1, 1)
        cnts = do_block(1, cnts)

        @pl.when(b0 + 3 < NBLK_PRE)
        def _():
            load_blk(b0 + 3, 1)

        return cnts

    z = jnp.int32(0)
    cnts = lax.fori_loop(0, NBLK_PRE // 2, pair, (z, z, z))

    tg = jnp.zeros((16,), jnp.int32)
    td = jnp.full((16,), N, jnp.int32)
    mall = jnp.ones((16,), jnp.bool_)
    for li in range(3):
        eref, dref = lists[li]
        for j in range(C // 16):
            plsc.store_compressed(eref.at[pl.ds(cnts[li] + j * 16, 16)],
                                  tg, mask=mall)
            plsc.store_compressed(dref.at[pl.ds(cnts[li] + j * 16, 16)],
                                  td, mask=mall)
        cbuf[...] = jnp.zeros((16,), jnp.int32) + cnts[li]
        pltpu.sync_copy(cbuf, cnts_hbm.at[li, w])
        pltpu.sync_copy(eref, eidx_hbm.at[li, w])
        pltpu.sync_copy(dref, edst_hbm.at[li, w])
    plsc.subcore_barrier()
    pltpu.sync_copy(dacc.at[pl.ds(rbase, ROWS_PER_TILE)],
                    degp_hbm.at[c, pl.ds(rbase, ROWS_PER_TILE)])


_sc_preprocess = pl.kernel(
    _pre_body,
    out_type=[
        jax.ShapeDtypeStruct((3, NW, CAP), jnp.int32),
        jax.ShapeDtypeStruct((3, NW, CAP), jnp.int32),
        jax.ShapeDtypeStruct((3, NW, 16), jnp.int32),
        jax.ShapeDtypeStruct((NC, NPAD, 16), jnp.float32),
    ],
    mesh=_mesh,
    scratch_types=[
        pltpu.VMEM((2, IB, C), jnp.int32),
        pltpu.VMEM((2, IB, C), jnp.int32),
        pltpu.VMEM((N + 16,), jnp.int32),
        pltpu.VMEM((CAP,), jnp.int32), pltpu.VMEM((CAP,), jnp.int32),
        pltpu.VMEM((CAP,), jnp.int32), pltpu.VMEM((CAP,), jnp.int32),
        pltpu.VMEM((CAP,), jnp.int32), pltpu.VMEM((CAP,), jnp.int32),
        pltpu.VMEM((16,), jnp.int32),
        pltpu.VMEM((C, 16), jnp.float32),
        pltpu.VMEM_SHARED((NPAD, 16), jnp.float32),
        pltpu.SemaphoreType.DMA((2,)),
        pltpu.SemaphoreType.DMA,
    ],
    compiler_params=pltpu.CompilerParams(use_tc_tiling_on_sc=False,
                                        needs_layout_passes=False))


def _sc_body(gidx_hbm, dst_hbm, cnt_hbm, table_hbm, zeros_hbm,
             out_hbm, gbuf, dbuf, rows, acc, gsem, isem, ssem, cbuf):
    c = lax.axis_index("c")
    s = lax.axis_index("s")
    w = s * NC + c
    rbase = s * ROWS_PER_TILE
    pltpu.sync_copy(cnt_hbm.at[w], cbuf)
    cnt = jnp.max(cbuf[...])
    nch = jnp.maximum(lax.div(cnt + (C - 1), C), 1)
    nblk = lax.div(nch + (IB - 1), IB)
    pltpu.sync_copy(zeros_hbm.at[pl.ds(rbase, ROWS_PER_TILE)],
                    acc.at[pl.ds(rbase, ROWS_PER_TILE)])
    plsc.subcore_barrier()

    pltpu.sync_copy(gidx_hbm.at[w, pl.ds(0, IB)], gbuf.at[0])
    pltpu.sync_copy(dst_hbm.at[w, pl.ds(0, IB)], dbuf.at[0])

    def issue(gslot, bslot, jj):
        pltpu.async_copy(table_hbm.at[gbuf.at[bslot, jj]], rows.at[gslot],
                         gsem.at[gslot])

    def blk_body(b, carry):
        bslot = lax.rem(b, 2)
        base = b * IB

        @pl.when(b > 0)
        def _():
            pltpu.make_async_copy(gidx_hbm.at[w, pl.ds(base, IB)],
                                  gbuf.at[bslot], isem.at[bslot]).wait()
            pltpu.make_async_copy(dst_hbm.at[w, pl.ds(base, IB)],
                                  dbuf.at[bslot], isem.at[bslot]).wait()

        @pl.when(b + 1 < nblk)
        def _():
            nbase = (b + 1) * IB
            pltpu.async_copy(gidx_hbm.at[w, pl.ds(nbase, IB)],
                             gbuf.at[1 - bslot], isem.at[1 - bslot])
            pltpu.async_copy(dst_hbm.at[w, pl.ds(nbase, IB)],
                             dbuf.at[1 - bslot], isem.at[1 - bslot])

        def scat_desc(gslot, jj):
            return pltpu.make_async_copy(rows.at[gslot],
                                         acc.at[dbuf.at[bslot, jj]],
                                         ssem.at[gslot])

        issue(lax.rem(base, 2), bslot, 0)
        for jj in range(IB):
            i = base + jj
            gslot = lax.rem(i, 2)

            @pl.when(i < nch)
            def _(jj=jj, i=i, gslot=gslot):
                pltpu.make_async_copy(table_hbm.at[gbuf.at[bslot, jj]],
                                      rows.at[gslot], gsem.at[gslot]).wait()
                if jj + 1 < IB:
                    @pl.when(i + 1 < nch)
                    def _():
                        if jj >= 1:
                            scat_desc(1 - gslot, jj - 1).wait()
                        issue(1 - gslot, bslot, jj + 1)
                pltpu.async_copy(rows.at[gslot], acc.at[dbuf.at[bslot, jj]],
                                 ssem.at[gslot], add=True)
        for jj in range(IB):
            i = base + jj
            gslot = lax.rem(i, 2)

            @pl.when(jnp.logical_and(
                i < nch,
                jnp.logical_or(jj >= IB - 2, i + 2 >= nch)))
            def _(jj=jj, gslot=gslot):
                scat_desc(gslot, jj).wait()
        return carry

    lax.fori_loop(0, nblk, blk_body, 0)
    plsc.subcore_barrier()
    pltpu.sync_copy(acc.at[pl.ds(rbase, ROWS_PER_TILE)],
                    out_hbm.at[c, pl.ds(rbase, ROWS_PER_TILE)])


_sc_scatter = pl.kernel(
    _sc_body,
    out_type=[jax.ShapeDtypeStruct((NC, NPAD, D), jnp.float32)],
    mesh=_mesh,
    scratch_types=[
        pltpu.VMEM((2, IB, C), jnp.int32),
        pltpu.VMEM((2, IB, C), jnp.int32),
        pltpu.VMEM((2, C, D), jnp.float32),
        pltpu.VMEM_SHARED((NPAD, D), jnp.float32),
        pltpu.SemaphoreType.DMA((2,)),
        pltpu.SemaphoreType.DMA((2,)),
        pltpu.SemaphoreType.DMA((2,)),
        pltpu.VMEM((16,), jnp.int32),
    ],
    compiler_params=pltpu.CompilerParams(use_tc_tiling_on_sc=False,
                                        needs_layout_passes=False))


def _po_body(hfin_hbm, poidx_hbm, out_hbm, pall, rows):
    c = lax.axis_index("c")
    s = lax.axis_index("s")
    w = s * NC + c
    pltpu.sync_copy(poidx_hbm.at[w], pall)
    for j in range(NCH_PO):
        pltpu.sync_copy(hfin_hbm.at[pall.at[j]], rows)
        pltpu.sync_copy(rows, out_hbm.at[pl.ds(w * NCH_PO * C_PO + j * C_PO,
                                               C_PO)])


_po_gather = pl.kernel(
    _po_body,
    out_type=jax.ShapeDtypeStruct((PO_PAD, D), jnp.float32),
    mesh=_mesh,
    scratch_types=[
        pltpu.VMEM((NCH_PO, C_PO), jnp.int32),
        pltpu.VMEM((C_PO, D), jnp.float32),
    ],
    compiler_params=pltpu.CompilerParams(use_tc_tiling_on_sc=False,
                                        needs_layout_passes=False),
)


def _update_body(l, acc0_ref, acc1_ref, deg0_ref, deg1_ref, lvl_ref, h_ref,
                 w1, b1, w2, b2, w3, b3, o_ref):
    i = pl.program_id(0)
    d = (deg0_ref[pl.ds(i * RB, RB), :] + deg1_ref[pl.ds(i * RB, RB), :])
    degcol = jnp.maximum(d[:, 0:1], 1.0)
    neigh = (acc0_ref[...] + acc1_ref[...]) / degcol
    upd = _mlp_block(neigh, w1, b1, w2, b2, w3, b3)
    lvl = lvl_ref[pl.ds(i * RB, RB), :]
    o_ref[...] = jnp.where(lvl == l, upd, h_ref[...])


def _update(l, acc0, acc1, deg0, deg1, lvl2d, hcur, w1, b1, w2, b2, w3, b3):
    return pl.pallas_call(
        functools.partial(_update_body, l),
        grid=(N // RB,),
        in_specs=[
            pl.BlockSpec((RB, D), lambda i: (i, 0)),
            pl.BlockSpec((RB, D), lambda i: (i, 0)),
            _full_spec((NPAD, 16)),
            _full_spec((NPAD, 16)),
            _full_spec((N, 1)),
            pl.BlockSpec((RB, D), lambda i: (i, 0)),
            _full_spec((D, H)), _full_spec((1, H)),
            _full_spec((H, H)), _full_spec((1, H)),
            _full_spec((H, D)), _full_spec((1, D)),
        ],
        out_specs=pl.BlockSpec((RB, D), lambda i: (i, 0)),
        out_shape=jax.ShapeDtypeStruct((N, D), jnp.float32),
    )(acc0, acc1, deg0, deg1, lvl2d, hcur, w1, b1, w2, b2, w3, b3)


def _upd_tab_body(l, acc0_ref, acc1_ref, deg0_ref, deg1_ref, lvl_ref, h_ref,
                  w1, b1, w2, b2, w3, b3, fw1, fb1, fw2, fb2, fw3, fb3,
                  o_ref, t_ref):
    i = pl.program_id(0)
    nb = N // RB
    ii = lax.rem(i, nb)
    d = (deg0_ref[pl.ds(ii * RB, RB), :] + deg1_ref[pl.ds(ii * RB, RB), :])
    degcol = jnp.maximum(d[:, 0:1], 1.0)
    neigh = (acc0_ref[...] + acc1_ref[...]) / degcol
    upd = _mlp_block(neigh, w1, b1, w2, b2, w3, b3)
    lvl = lvl_ref[pl.ds(ii * RB, RB), :]
    hnew = jnp.where(lvl == l, upd, h_ref[...])
    o_ref[...] = hnew

    @pl.when(i < nb)
    def _():
        t_ref[...] = hnew

    @pl.when(i >= nb)
    def _():
        t_ref[...] = _mlp_block(hnew, fw1, fb1, fw2, fb2, fw3, fb3)


def _update_and_table(l, acc0, acc1, deg0, deg1, lvl2d, hcur,
                      w1, b1, w2, b2, w3, b3, fw1, fb1, fw2, fb2, fw3, fb3):
    nb = N // RB
    blk = pl.BlockSpec((RB, D), lambda i: (lax.rem(i, N // RB), 0))
    return pl.pallas_call(
        functools.partial(_upd_tab_body, l),
        grid=(2 * nb,),
        in_specs=[
            blk, blk,
            _full_spec((NPAD, 16)), _full_spec((NPAD, 16)),
            _full_spec((N, 1)),
            blk,
            _full_spec((D, H)), _full_spec((1, H)),
            _full_spec((H, H)), _full_spec((1, H)),
            _full_spec((H, D)), _full_spec((1, D)),
            _full_spec((D, H)), _full_spec((1, H)),
            _full_spec((H, H)), _full_spec((1, H)),
            _full_spec((H, D)), _full_spec((1, D)),
        ],
        out_specs=[
            pl.BlockSpec((RB, D), lambda i: (lax.rem(i, N // RB), 0)),
            pl.BlockSpec((RB, D), lambda i: (i, 0)),
        ],
        out_shape=[
            jax.ShapeDtypeStruct((N, D), jnp.float32),
            jax.ShapeDtypeStruct((2 * N, D), jnp.float32),
        ],
    )(acc0, acc1, deg0, deg1, lvl2d, hcur, w1, b1, w2, b2, w3, b3,
      fw1, fb1, fw2, fb2, fw3, fb3)


def _pr_body(x_ref, w1, b1, w2, b2, w3, b3, o_ref):
    o_ref[...] = _mlp_block(x_ref[...], w1, b1, w2, b2, w3, b3)


def _pr_mlp(x, w1, b1, w2, b2, w3, b3):
    rb = PO_PAD // 8
    return pl.pallas_call(
        _pr_body,
        grid=(8,),
        in_specs=[
            pl.BlockSpec((rb, D), lambda i: (i, 0)),
            _full_spec((D, H)), _full_spec((1, H)),
            _full_spec((H, H)), _full_spec((1, H)),
            _full_spec((H, D)), _full_spec((1, D)),
        ],
        out_specs=pl.BlockSpec((rb, D), lambda i: (i, 0)),
        out_shape=jax.ShapeDtypeStruct((PO_PAD, D), jnp.float32),
    )(x, w1, b1, w2, b2, w3, b3)


def kernel(h, edge_index, edge_r, node_level, PO_mask,
           fi_w1, fi_b1, fi_w2, fi_b2, fi_w3, fi_b3,
           fa_w1, fa_b1, fa_w2, fa_b2, fa_w3, fa_b3,
           pr_w1, pr_b1, pr_w2, pr_b2, pr_w3, pr_b3):
    src = edge_index[0]
    dst = edge_index[1]
    pad = EPAD - E
    src_p = jnp.concatenate([src, jnp.zeros((pad,), jnp.int32)])
    r_p = jnp.concatenate([edge_r, jnp.zeros((pad,), jnp.int32)])
    dst_p = jnp.concatenate([dst, jnp.full((pad,), N, jnp.int32)])
    gidx2 = _build_gidx(src_p, r_p)
    dst2 = dst_p.reshape(NW, NCH, C)
    zeros = jnp.zeros((NPAD, D), jnp.float32)
    zeros16 = jnp.zeros((NPAD, 16), jnp.float32)
    ones16 = jnp.ones((C, 16), jnp.float32)
    lvl2d = node_level.reshape(N, 1)
    nlpad = jnp.concatenate([node_level, jnp.zeros((16,), jnp.int32)])

    eidx3, edst3, cnts3, degp = _sc_preprocess(gidx2, dst2, nlpad, zeros16,
                                               ones16)
    deg0, deg1 = degp[0], degp[1]

    fi_b1r, fi_b2r, fi_b3r = (fi_b1.reshape(1, H), fi_b2.reshape(1, H),
                              fi_b3.reshape(1, D))
    fa_b1r, fa_b2r, fa_b3r = (fa_b1.reshape(1, H), fa_b2.reshape(1, H),
                              fa_b3.reshape(1, D))
    pr_b1r, pr_b2r, pr_b3r = (pr_b1.reshape(1, H), pr_b2.reshape(1, H),
                              pr_b3.reshape(1, D))

    hcur = h
    T = _build_table(hcur, fi_w1, fi_b1r, fi_w2, fi_b2r, fi_w3, fi_b3r)
    for l in range(1, NLEVELS):
        (parts,) = _sc_scatter(eidx3[l - 1].reshape(NW, NCH_CAP, C),
                               edst3[l - 1].reshape(NW, NCH_CAP, C),
                               cnts3[l - 1], T, zeros)
        if l < NLEVELS - 1:
            hcur, T = _update_and_table(
                l, parts[0], parts[1], deg0, deg1, lvl2d, hcur,
                fa_w1, fa_b1r, fa_w2, fa_b2r, fa_w3, fa_b3r,
                fi_w1, fi_b1r, fi_w2, fi_b2r, fi_w3, fi_b3r)
        else:
            hcur = _update(l, parts[0], parts[1], deg0, deg1, lvl2d, hcur,
                           fa_w1, fa_b1r, fa_w2, fa_b2r, fa_w3, fa_b3r)

    po_p = jnp.concatenate(
        [PO_mask, jnp.zeros((PO_PAD - NPO,), jnp.int32)]).reshape(
            NW, NCH_PO, C_PO)
    rows = _po_gather(hcur, po_p)
    out = _pr_mlp(rows, pr_w1, pr_b1r, pr_w2, pr_b2r, pr_w3, pr_b3r)
    return out[:NPO]
